# Initial kernel scaffold; baseline (speedup 1.0000x reference)
import jax
import jax.numpy as jnp
from jax.experimental import pallas as pl


def kernel(feat_s8, feat_s16, feat_s32):
    raise NotImplementedError("write your pallas kernel here")


# speedup vs baseline: 1.5295x; 1.5295x over previous
import jax
import jax.numpy as jnp
import numpy as np
from jax.experimental import pallas as pl

IMG_SIZE = 640
STRIDES = (8, 16, 32)
A = 5
NO = 85
RATIOS = (1.0, 2.0, 0.5, 3.0, 1.0 / 3.0)
B = 8
FS = (80, 40, 20)
HW = (6400, 1600, 400)
TSP = (1280, 1600, 400)
ROWOFF = (0, 32000, 40000)
TOTAL = 42000


def _decode_tile(x, a_id, level, t_id, tsp):
    fs = FS[level]
    base = 4.0 * STRIDES[level] / IMG_SIZE
    sr = [float(np.sqrt(r)) for r in RATIOS]
    sqrt_r = jnp.where(a_id == 0, sr[0],
             jnp.where(a_id == 1, sr[1],
             jnp.where(a_id == 2, sr[2],
             jnp.where(a_id == 3, sr[3], sr[4]))))
    w = base * sqrt_r
    h = base / sqrt_r

    p = t_id * tsp + jax.lax.broadcasted_iota(jnp.int32, (1, tsp), 1)
    xi = jax.lax.rem(p, fs)
    yi = jax.lax.div(p, fs)
    inv_fs = 1.0 / fs
    cx = (xi.astype(jnp.float32) + 0.5) * inv_fs
    cy = (yi.astype(jnp.float32) + 0.5) * inv_fs

    row = jax.lax.broadcasted_iota(jnp.int32, (NO, tsp), 0)
    cls_mask = row >= 5

    neg = jnp.float32(-1e30)
    xm = jnp.where(cls_mask, x, neg)
    m = jnp.max(xm, axis=0, keepdims=True)
    e = jnp.exp(x - m)
    s = jnp.sum(jnp.where(cls_mask, e, 0.0), axis=0, keepdims=True)
    cls = e / s

    scale = jnp.where((row == 0) | (row == 2), w, h)
    off = jnp.where(row == 0, cx, cy)
    xy = x * scale + off
    wh = jnp.exp(x) * scale
    obj = jax.nn.sigmoid(x)

    y = jnp.where(row < 2, xy,
        jnp.where(row < 4, wh,
        jnp.where(row == 4, obj, cls)))
    return y.T


def _make_body(level):
    tsp = TSP[level]

    def body(f, o, *, _prev=None):
        a_id = pl.program_id(1)
        t_id = pl.program_id(2)
        o[0] = _decode_tile(f[0, 0], a_id, level, t_id, tsp)

    def body_aliased(f, prev, o):
        del prev
        body(f, o)

    return body if level == 0 else body_aliased


def _level_call(level, feat, prev_out):
    tsp = TSP[level]
    tiles = HW[level] // tsp
    rowblk0 = ROWOFF[level] // tsp
    f = feat.reshape(B, A, NO, HW[level])

    in_specs = [
        pl.BlockSpec((1, 1, NO, tsp), lambda b, a, t: (b, a, 0, t)),
    ]
    operands = [f]
    kwargs = {}
    if prev_out is not None:
        in_specs.append(pl.BlockSpec((1, 8, NO), lambda b, a, t: (0, 0, 0)))
        operands.append(prev_out)
        kwargs['input_output_aliases'] = {1: 0}

    return pl.pallas_call(
        _make_body(level),
        grid=(B, A, tiles),
        in_specs=in_specs,
        out_specs=pl.BlockSpec(
            (1, tsp, NO), lambda b, a, t: (b, rowblk0 + a * tiles + t, 0)
        ),
        out_shape=jax.ShapeDtypeStruct((B, TOTAL, NO), jnp.float32),
        **kwargs,
    )(*operands)


def kernel(feat_s8, feat_s16, feat_s32):
    out = _level_call(0, feat_s8, None)
    out = _level_call(1, feat_s16, out)
    out = _level_call(2, feat_s32, out)
    return out


# speedup vs baseline: 1.7302x; 1.1312x over previous
import jax
import jax.numpy as jnp
import numpy as np
from jax.experimental import pallas as pl

IMG_SIZE = 640
STRIDES = (8, 16, 32)
A = 5
NO = 85
RATIOS = (1.0, 2.0, 0.5, 3.0, 1.0 / 3.0)
B = 8
FS = (80, 40, 20)
HW = (6400, 1600, 400)
TSP = (6400, 1600, 400)
ROWOFF = (0, 32000, 40000)
TOTAL = 42000


def _decode_tile(x, a_id, level, t_id, tsp):
    fs = FS[level]
    base = 4.0 * STRIDES[level] / IMG_SIZE
    sr = [float(np.sqrt(r)) for r in RATIOS]
    sqrt_r = jnp.where(a_id == 0, sr[0],
             jnp.where(a_id == 1, sr[1],
             jnp.where(a_id == 2, sr[2],
             jnp.where(a_id == 3, sr[3], sr[4]))))
    w = base * sqrt_r
    h = base / sqrt_r

    p = t_id * tsp + jax.lax.broadcasted_iota(jnp.int32, (1, tsp), 1)
    xi = jax.lax.rem(p, fs)
    yi = jax.lax.div(p, fs)
    inv_fs = 1.0 / fs
    cx = (xi.astype(jnp.float32) + 0.5) * inv_fs
    cy = (yi.astype(jnp.float32) + 0.5) * inv_fs

    row = jax.lax.broadcasted_iota(jnp.int32, (NO, tsp), 0)
    cls_mask = row >= 5

    neg = jnp.float32(-1e30)
    xm = jnp.where(cls_mask, x, neg)
    m = jnp.max(xm, axis=0, keepdims=True)
    e = jnp.exp(x - m)
    s = jnp.sum(jnp.where(cls_mask, e, 0.0), axis=0, keepdims=True)
    cls = e / s

    scale = jnp.where((row == 0) | (row == 2), w, h)
    off = jnp.where(row == 0, cx, cy)
    xy = x * scale + off
    wh = jnp.exp(x) * scale
    obj = jax.nn.sigmoid(x)

    y = jnp.where(row < 2, xy,
        jnp.where(row < 4, wh,
        jnp.where(row == 4, obj, cls)))
    return y.T


def _make_body(level):
    tsp = TSP[level]

    def body(f, o, *, _prev=None):
        a_id = pl.program_id(1)
        t_id = pl.program_id(2)
        o[0] = _decode_tile(f[0, 0], a_id, level, t_id, tsp)

    def body_aliased(f, prev, o):
        del prev
        body(f, o)

    return body if level == 0 else body_aliased


def _level_call(level, feat, prev_out):
    tsp = TSP[level]
    tiles = HW[level] // tsp
    rowblk0 = ROWOFF[level] // tsp
    f = feat.reshape(B, A, NO, HW[level])

    in_specs = [
        pl.BlockSpec((1, 1, NO, tsp), lambda b, a, t: (b, a, 0, t)),
    ]
    operands = [f]
    kwargs = {}
    if prev_out is not None:
        in_specs.append(pl.BlockSpec((1, 8, NO), lambda b, a, t: (0, 0, 0)))
        operands.append(prev_out)
        kwargs['input_output_aliases'] = {1: 0}

    return pl.pallas_call(
        _make_body(level),
        grid=(B, A, tiles),
        in_specs=in_specs,
        out_specs=pl.BlockSpec(
            (1, tsp, NO), lambda b, a, t: (b, rowblk0 + a * tiles + t, 0)
        ),
        out_shape=jax.ShapeDtypeStruct((B, TOTAL, NO), jnp.float32),
        **kwargs,
    )(*operands)


def kernel(feat_s8, feat_s16, feat_s32):
    out = _level_call(0, feat_s8, None)
    out = _level_call(1, feat_s16, out)
    out = _level_call(2, feat_s32, out)
    return out
